# Initial kernel scaffold; baseline (speedup 1.0000x reference)
import jax
import jax.numpy as jnp
from jax.experimental import pallas as pl


def kernel(text_feats, text, attenscore, fc_w, fc_b, mlp_w0, mlp_b0, mlp_w1, mlp_b1, bn0_gamma, bn0_beta):
    raise NotImplementedError("write your pallas kernel here")


# speedup vs baseline: 2.9250x; 2.9250x over previous
import functools

import jax
import jax.numpy as jnp
from jax import lax
from jax.experimental import pallas as pl
from jax.experimental.pallas import tpu as pltpu
from jax.experimental.pallas import tpu_sc as plsc

_B = 4
_L = 4096
_D = 512
_E = 1024
_H = 512
_K = (_L - 2) // 2
_KP = _K + 1
_NTOT = _B * _K

_RC = 32
_LANES = 128


def _shift_lanes(x, s):
    return jnp.concatenate(
        [jnp.zeros(x.shape[:-1] + (s,), x.dtype), x[..., :-s]], axis=-1)


def _shift_rows(x, s):
    return jnp.concatenate(
        [jnp.zeros((s,) + x.shape[1:], x.dtype), x[:-s]], axis=0)


def _eprefix(m):
    x = m.astype(jnp.int32)
    inc = x
    for s in (1, 2, 4, 8, 16, 32, 64):
        inc = inc + _shift_lanes(inc, s)
    rowtot = inc[:, _LANES - 1:_LANES]
    rincl = rowtot
    for s in (1, 2, 4, 8, 16):
        rincl = rincl + _shift_rows(rincl, s)
    roff = rincl - rowtot
    return inc - x + roff


def _seg_total(x_flat, seg4, ones128):
    return jnp.dot(jnp.dot(seg4, x_flat, preferred_element_type=jnp.float32),
                   ones128, preferred_element_type=jnp.float32)


def _nth_largest4(ikey_flat, n_f, seg4, ones128):
    t = jnp.full((4, _LANES), -2147483648, jnp.int32)
    for j in range(31, -1, -1):
        if j == 31:
            cand = jnp.zeros((4, _LANES), jnp.int32)
        else:
            cand = t + jnp.int32(1 << j)
        cand_exp = jnp.broadcast_to(cand[:, None, :],
                                    (_B, _RC, _LANES)).reshape(_B * _RC, _LANES)
        tot = _seg_total((ikey_flat >= cand_exp).astype(jnp.float32),
                         seg4, ones128)
        t = jnp.where(tot >= n_f, cand, t)
    return t


def _rank_body(text_ref, atts_ref, dst_ref, vlen_ref, att_s,
               sem0, sem1, sem2, sem3):
    idx2 = (lax.broadcasted_iota(jnp.int32, (_RC, _LANES), 0) * _LANES
            + lax.broadcasted_iota(jnp.int32, (_RC, _LANES), 1))
    ones128 = jnp.ones((_LANES, _LANES), jnp.float32)
    seg4 = (lax.broadcasted_iota(jnp.int32, (_B, _B * _RC), 1) // _RC
            == lax.broadcasted_iota(jnp.int32, (_B, _B * _RC), 0)
            ).astype(jnp.float32)

    text_all = text_ref[...]
    sems = (sem0, sem1, sem2, sem3)
    copies = []
    argms = []
    for b in range(_B):
        t2 = text_all[b]
        maxv = jnp.max(t2)
        argm = jnp.min(jnp.where(t2 == maxv, idx2, _L))
        cp = pltpu.make_async_copy(atts_ref.at[b, argm], att_s.at[b], sems[b])
        cp.start()
        copies.append(cp)
        argms.append(argm)

    mask_all = (text_all != 0)
    mask_flat = mask_all.reshape(_B * _RC, _LANES).astype(jnp.float32)
    lengths4 = _seg_total(mask_flat, seg4, ones128).astype(jnp.int32) - 2
    vl4 = jnp.minimum(lengths4, _K)
    vl04 = jnp.maximum(vl4, 0)
    vlen_ref[...] = jnp.broadcast_to(vl4[:, None, :], (_B, 8, _LANES))

    for cp in copies:
        cp.wait()

    att_rows = []
    for b in range(_B):
        a2 = att_s[b]
        a2 = jnp.where((idx2 == argms[b]) | (idx2 == 0), -1.0, a2)
        a2 = a2 * mask_all[b].astype(jnp.float32) + 0.0
        att_rows.append(a2)
    att_all = jnp.concatenate([a[None] for a in att_rows], axis=0)

    bits = lax.bitcast_convert_type(att_all, jnp.int32)
    mag = bits & jnp.int32(0x7FFFFFFF)
    ikey_all = jnp.where(bits < 0, -mag, mag)
    ikey_flat = ikey_all.reshape(_B * _RC, _LANES)

    kf = jnp.full((4, _LANES), float(_K), jnp.float32)
    tk4 = _nth_largest4(ikey_flat, kf, seg4, ones128)
    tv4 = _nth_largest4(ikey_flat, vl04.astype(jnp.float32), seg4, ones128)

    gtk_flat = (ikey_flat > jnp.broadcast_to(
        tk4[:, None, :], (_B, _RC, _LANES)).reshape(_B * _RC, _LANES))
    gtv_flat = (ikey_flat > jnp.broadcast_to(
        tv4[:, None, :], (_B, _RC, _LANES)).reshape(_B * _RC, _LANES))
    nk4 = _K - _seg_total(gtk_flat.astype(jnp.float32),
                          seg4, ones128).astype(jnp.int32)
    nv4 = vl04 - _seg_total(gtv_flat.astype(jnp.float32),
                            seg4, ones128).astype(jnp.int32)

    gtk_all = gtk_flat.reshape(_B, _RC, _LANES)
    gtv_all = gtv_flat.reshape(_B, _RC, _LANES)
    for b in range(_B):
        ikey = ikey_all[b]
        eqk = ikey == tk4[b:b + 1, :]
        selk = gtk_all[b] | (eqk & (_eprefix(eqk) < nk4[b:b + 1, :]))
        eqv = ikey == tv4[b:b + 1, :]
        validm = gtv_all[b] | (eqv & (_eprefix(eqv) < nv4[b:b + 1, :]))
        smask = selk & (~validm)
        dst = jnp.where(validm, _eprefix(validm),
                        jnp.where(smask, vl04[b:b + 1, :] + _eprefix(smask),
                                  jnp.int32(_K)))
        dst_ref[b] = dst


def _stage_a(text2, atts4):
    return pl.pallas_call(
        _rank_body,
        grid=(1,),
        in_specs=[
            pl.BlockSpec((_B, _RC, _LANES), lambda i: (0, 0, 0)),
            pl.BlockSpec(memory_space=pl.ANY),
        ],
        out_specs=[
            pl.BlockSpec((_B, _RC, _LANES), lambda i: (0, 0, 0)),
            pl.BlockSpec((_B, 8, _LANES), lambda i: (0, 0, 0)),
        ],
        out_shape=[
            jax.ShapeDtypeStruct((_B, _RC, _LANES), jnp.int32),
            jax.ShapeDtypeStruct((_B, 8, _LANES), jnp.int32),
        ],
        scratch_shapes=[
            pltpu.VMEM((_B, _RC, _LANES), jnp.float32),
            pltpu.SemaphoreType.DMA,
            pltpu.SemaphoreType.DMA,
            pltpu.SemaphoreType.DMA,
            pltpu.SemaphoreType.DMA,
        ],
        compiler_params=pltpu.CompilerParams(
            dimension_semantics=("arbitrary",),
        ),
    )(text2, atts4)


_NW = 32
_RPW = (_B * _L) // _NW
_CH = 64
_NCHUNK = _RPW // _CH


_SPR = (_B * _L) // 16
_SLW = (_B * _KP) // _NW
_GR = 64


def _sc_scatter_body(dst_hbm, feats_hbm, out_hbm,
                     dstv, vals3, didx3, inv_v, row0_v, row1_v, inv_sh,
                     gsem0, gsem1, wsem0, wsem1):
    c = lax.axis_index("c")
    s = lax.axis_index("s")
    w = s * 2 + c

    r0 = s * _SPR
    pltpu.sync_copy(dst_hbm.at[pl.ds(r0, _SPR)], dstv)
    iota16 = lax.iota(jnp.int32, 16)

    @pl.when(s == 0)
    def _():
        for j in range(_LANES // 16):
            sl = pl.ds(j * 16, 16)
            vals3[0, sl] = jnp.zeros((16,), jnp.int32)
            didx3[0, sl] = jnp.minimum(iota16, 3) * _KP + _K
        pltpu.sync_copy(vals3.at[0], inv_sh.at[didx3.at[0]])

    for g in range(_SPR // _LANES):
        gb = r0 + g * _LANES
        dbase = (gb // _L) * _KP
        for j in range(_LANES // 16):
            sl = pl.ds(j * 16, 16)
            rowid = iota16 + (gb + j * 16)
            dd = dstv[pl.ds(g * _LANES + j * 16, 16)]
            vals3[g, sl] = rowid
            didx3[g, sl] = jnp.where(dd < _K, dd + dbase,
                                     rowid + (_B * _KP))
        pltpu.sync_copy(vals3.at[g], inv_sh.at[didx3.at[g]])
    plsc.subcore_barrier()

    slot0 = w * _SLW
    pltpu.sync_copy(inv_sh.at[pl.ds(slot0, _SLW)], inv_v)
    bufs = (row0_v, row1_v)
    gsems = (gsem0, gsem1)
    wsems = (wsem0, wsem1)
    nr = _SLW // _GR
    gets = [None] * nr
    puts = [None] * nr
    gets[0] = pltpu.async_copy(
        feats_hbm.at[inv_v.at[pl.ds(0, _GR)]], bufs[0], gsems[0])
    for r in range(nr):
        buf = r % 2
        if r + 1 < nr:
            if r - 1 >= 0:
                puts[r - 1].wait()
            gets[r + 1] = pltpu.async_copy(
                feats_hbm.at[inv_v.at[pl.ds((r + 1) * _GR, _GR)]],
                bufs[1 - buf], gsems[1 - buf])
        gets[r].wait()
        puts[r] = pltpu.async_copy(
            bufs[buf], out_hbm.at[pl.ds(slot0 + r * _GR, _GR)], wsems[buf])
    puts[nr - 2].wait()
    puts[nr - 1].wait()


@functools.cache
def _make_stage_b():
    return functools.partial(
        pl.kernel,
        out_type=jax.ShapeDtypeStruct((_B * _KP, _D), jnp.float32),
        mesh=plsc.VectorSubcoreMesh(core_axis_name="c", subcore_axis_name="s"),
        scratch_types=[
            pltpu.VMEM((_SPR,), jnp.int32),
            pltpu.VMEM((_SPR // _LANES, _LANES), jnp.int32),
            pltpu.VMEM((_SPR // _LANES, _LANES), jnp.int32),
            pltpu.VMEM((_SLW,), jnp.int32),
            pltpu.VMEM((_GR, _D), jnp.float32),
            pltpu.VMEM((_GR, _D), jnp.float32),
            pltpu.VMEM_SHARED((_B * _KP + _B * _L,), jnp.int32),
            pltpu.SemaphoreType.DMA,
            pltpu.SemaphoreType.DMA,
            pltpu.SemaphoreType.DMA,
            pltpu.SemaphoreType.DMA,
        ],
    )(_sc_scatter_body)


def _stage_b(dst_flat, feats_flat):
    return _make_stage_b()(dst_flat, feats_flat)


_RB = 512
_NBLK = (_B * _KP) // _RB
_RPB = _KP // _RB


def _c_body(base_ref, vlen_ref, w0t_ref, b0_ref, w1t_ref, b1_ref,
            fct_ref, fcb_ref, g_ref, bt_ref, out_ref,
            xh_s, h_s, acc_s, stats_s):
    i = pl.program_id(0)

    @pl.when(i == 0)
    def _():
        acc_s[...] = jnp.zeros_like(acc_s)

    @pl.when(i < _NBLK)
    def _():
        rows = base_ref[...]
        ss = jnp.sum(rows * rows, axis=1, keepdims=True)
        xh32 = rows / (jnp.sqrt(ss) + 1e-8)
        xh_s[pl.ds(i * _RB, _RB), :] = xh32

        h32 = jnp.dot(xh32, w0t_ref[...],
                      preferred_element_type=jnp.float32) + b0_ref[...]
        h_s[pl.ds(i * _RB, _RB), :] = h32

        pos = i * _RB + lax.broadcasted_iota(jnp.int32, (_RB, _H), 0)
        validrow = (pos % _KP) < _K
        hv = jnp.where(validrow, h32, 0.0)
        acc_s[0:1, :] = acc_s[0:1, :] + jnp.sum(hv, axis=0, keepdims=True)
        acc_s[1:2, :] = acc_s[1:2, :] + jnp.sum(hv * hv, axis=0, keepdims=True)

        @pl.when(i == _NBLK - 1)
        def _():
            mu = acc_s[0:1, :] / _NTOT
            var = acc_s[1:2, :] / _NTOT - mu * mu
            stats_s[0:1, :] = mu
            stats_s[1:2, :] = var

    @pl.when(i >= _NBLK)
    def _():
        r = i - _NBLK
        b = r // _RPB
        rb = r % _RPB
        vl = vlen_ref[b, 0]

        mu = stats_s[0:1, :]
        var = stats_s[1:2, :]
        h32 = h_s[pl.ds(r * _RB, _RB), :]
        hn = (h32 - mu) / jnp.sqrt(var + 1e-5) * g_ref[...] + bt_ref[...]
        r32 = jnp.maximum(hn, 0.0)

        cap = jnp.dot(r32, w1t_ref[...],
                      preferred_element_type=jnp.float32) + b1_ref[...]
        loc = jnp.dot(xh_s[pl.ds(r * _RB, _RB), :], fct_ref[...],
                      preferred_element_type=jnp.float32) + fcb_ref[...]
        local = loc + cap

        pos = rb * _RB + lax.broadcasted_iota(jnp.int32, (_RB, _E), 0)
        valid = pos < vl
        masked = jnp.where(valid, local, -jnp.inf)
        mx = jnp.max(masked, axis=0, keepdims=True).reshape(1, 1, _E)

        @pl.when(rb == 0)
        def _():
            out_ref[...] = mx

        @pl.when(rb != 0)
        def _():
            out_ref[...] = jnp.maximum(out_ref[...], mx)


def _stage_c(base, vlen, w0t, b0c, w1t, b1c, fct, fcbc, g2, bt2):
    nsteps = 2 * _NBLK
    return pl.pallas_call(
        _c_body,
        grid=(nsteps,),
        in_specs=[
            pl.BlockSpec((_RB, _D), lambda i: (jnp.minimum(i, _NBLK - 1), 0)),
            pl.BlockSpec((_B, 1), lambda i: (0, 0), memory_space=pltpu.SMEM),
            pl.BlockSpec((_D, _H), lambda i: (0, 0)),
            pl.BlockSpec((1, _H), lambda i: (0, 0)),
            pl.BlockSpec((_H, _E), lambda i: (0, 0)),
            pl.BlockSpec((1, _E), lambda i: (0, 0)),
            pl.BlockSpec((_D, _E), lambda i: (0, 0)),
            pl.BlockSpec((1, _E), lambda i: (0, 0)),
            pl.BlockSpec((1, _H), lambda i: (0, 0)),
            pl.BlockSpec((1, _H), lambda i: (0, 0)),
        ],
        out_specs=pl.BlockSpec(
            (1, 1, _E),
            lambda i: (jnp.maximum(i - _NBLK, 0) // _RPB, 0, 0)),
        out_shape=jax.ShapeDtypeStruct((_B, 1, _E), jnp.float32),
        scratch_shapes=[
            pltpu.VMEM((_B * _KP, _D), jnp.float32),
            pltpu.VMEM((_B * _KP, _H), jnp.float32),
            pltpu.VMEM((8, _H), jnp.float32),
            pltpu.VMEM((8, _H), jnp.float32),
        ],
        compiler_params=pltpu.CompilerParams(
            dimension_semantics=("arbitrary",),
        ),
    )(base, vlen, w0t, b0c, w1t, b1c, fct, fcbc, g2, bt2)


def kernel(text_feats, text, attenscore, fc_w, fc_b, mlp_w0, mlp_b0,
           mlp_w1, mlp_b1, bn0_gamma, bn0_beta):
    text2 = text.reshape(_B, _RC, _LANES)
    atts4 = attenscore.reshape(_B, _L, _RC, _LANES)
    dst, vlenb = _stage_a(text2, atts4)

    dst_flat = dst.reshape(_B * _L)
    feats_flat = text_feats.reshape(_B * _L, _D)
    base = _stage_b(dst_flat, feats_flat)

    f32 = jnp.float32
    f16 = jnp.float16
    w0t = mlp_w0.astype(f16).astype(f32).T
    b0c = mlp_b0.astype(f16).astype(f32).reshape(1, _H)
    w1t = mlp_w1.astype(f16).astype(f32).T
    b1c = mlp_b1.astype(f16).astype(f32).reshape(1, _E)
    fct = fc_w.astype(f16).astype(f32).T
    fcbc = fc_b.astype(f16).astype(f32).reshape(1, _E)
    g2 = bn0_gamma.reshape(1, _H)
    bt2 = bn0_beta.reshape(1, _H)

    vlen = vlenb[:, 0, 0].reshape(_B, 1)
    pooled = _stage_c(base, vlen, w0t, b0c, w1t, b1c, fct, fcbc, g2, bt2)
    return pooled.reshape(_B, _E)
